# baseline (speedup 1.0000x reference)
import functools

import jax
import jax.numpy as jnp
from jax import lax
from jax.experimental import pallas as pl
from jax.experimental.pallas import tpu as pltpu
from jax.experimental.pallas import tpu_sc as plsc

BATCH = 16384
L = 16


def _sigmoid(x):
    return 1.0 / (1.0 + jnp.exp(-x))


def _flat(t, r, c):
    t2 = jnp.reshape(t, (r, c))
    t2 = lax.optimization_barrier(t2)
    return jnp.reshape(t2, (r * c,))


def _body(nc, bpw, user_hbm, item_hbm, theta_hbm, a_hbm, b_hbm, c_hbm,
          out_hbm, uidx_v, iidx_v, th_v, a_v, b_v, c_v, out_v, isem, gsem):
    wid = lax.axis_index("s") * nc + lax.axis_index("c")
    base = wid * bpw
    ciu = pltpu.async_copy(user_hbm.at[pl.ds(base, bpw)], uidx_v, isem)
    cii = pltpu.async_copy(item_hbm.at[pl.ds(base, bpw)], iidx_v, isem)
    ciu.wait()
    cii.wait()
    cth = pltpu.async_copy(theta_hbm.at[uidx_v], th_v, gsem)
    ca = pltpu.async_copy(a_hbm.at[iidx_v], a_v, gsem)
    cb = pltpu.async_copy(b_hbm.at[iidx_v], b_v, gsem)
    cc = pltpu.async_copy(c_hbm.at[iidx_v], c_v, gsem)
    cth.wait()
    ca.wait()
    cb.wait()
    cc.wait()

    def step(i, carry):
        sl = pl.ds(i * L, L)
        th = _sigmoid(th_v[sl]) - 0.5
        a = _sigmoid(a_v[sl])
        b = _sigmoid(b_v[sl]) - 0.5
        c = _sigmoid(c_v[sl])
        out_v[sl] = c + (1.0 - c) * _sigmoid(1.702 * a * (th - b))
        return carry

    lax.fori_loop(0, bpw // L, step, 0)
    pltpu.sync_copy(out_v, out_hbm.at[pl.ds(base, bpw)])


def kernel(user, item, theta_table, a_table, b_table, c_table):
    info = plsc.get_sparse_core_info()
    nc, ns = info.num_cores, info.num_subcores
    bpw = BATCH // (nc * ns)
    mesh = plsc.VectorSubcoreMesh(core_axis_name="c", subcore_axis_name="s")
    k = pl.kernel(
        functools.partial(_body, nc, bpw),
        mesh=mesh,
        out_type=jax.ShapeDtypeStruct((BATCH,), jnp.float32),
        scratch_types=[
            pltpu.VMEM((bpw,), jnp.int32),
            pltpu.VMEM((bpw,), jnp.int32),
            pltpu.VMEM((bpw,), jnp.float32),
            pltpu.VMEM((bpw,), jnp.float32),
            pltpu.VMEM((bpw,), jnp.float32),
            pltpu.VMEM((bpw,), jnp.float32),
            pltpu.VMEM((bpw,), jnp.float32),
            pltpu.SemaphoreType.DMA,
            pltpu.SemaphoreType.DMA,
        ],
    )
    return k(user, item, _flat(theta_table, 1000, 1000),
             _flat(a_table, 100, 1000),
             _flat(b_table, 100, 1000),
             _flat(c_table, 100, 1000))


# speedup vs baseline: 1.1317x; 1.1317x over previous
import functools

import jax
import jax.numpy as jnp
from jax import lax
from jax.experimental import pallas as pl
from jax.experimental.pallas import tpu as pltpu
from jax.experimental.pallas import tpu_sc as plsc

BATCH = 16384
L = 16


def _sigmoid(x):
    return 1.0 / (1.0 + jnp.exp(-x))


def _body(nc, bpw, user_hbm, item_hbm, theta_hbm, a_hbm, b_hbm, c_hbm,
          out_hbm, uidx_v, iidx_v, th_v, a_v, b_v, c_v, out_v, isem, gsem):
    wid = lax.axis_index("s") * nc + lax.axis_index("c")
    base = wid * bpw
    ciu = pltpu.async_copy(user_hbm.at[pl.ds(base, bpw)], uidx_v, isem)
    cii = pltpu.async_copy(item_hbm.at[pl.ds(base, bpw)], iidx_v, isem)
    ciu.wait()
    cii.wait()
    cth = pltpu.async_copy(theta_hbm.at[uidx_v], th_v, gsem)
    ca = pltpu.async_copy(a_hbm.at[iidx_v], a_v, gsem)
    cb = pltpu.async_copy(b_hbm.at[iidx_v], b_v, gsem)
    cc = pltpu.async_copy(c_hbm.at[iidx_v], c_v, gsem)
    cth.wait()
    ca.wait()
    cb.wait()
    cc.wait()

    def step(i, carry):
        sl = pl.ds(i * L, L)
        th = _sigmoid(th_v[sl]) - 0.5
        a = _sigmoid(a_v[sl])
        b = _sigmoid(b_v[sl]) - 0.5
        c = _sigmoid(c_v[sl])
        out_v[sl] = c + (1.0 - c) * _sigmoid(1.702 * a * (th - b))
        return carry

    lax.fori_loop(0, bpw // L, step, 0)
    pltpu.sync_copy(out_v, out_hbm.at[pl.ds(base, bpw)])


def kernel(user, item, theta_table, a_table, b_table, c_table):
    info = plsc.get_sparse_core_info()
    nc, ns = info.num_cores, info.num_subcores
    bpw = BATCH // (nc * ns)
    mesh = plsc.VectorSubcoreMesh(core_axis_name="c", subcore_axis_name="s")
    k = pl.kernel(
        functools.partial(_body, nc, bpw),
        mesh=mesh,
        out_type=jax.ShapeDtypeStruct((BATCH,), jnp.float32),
        scratch_types=[
            pltpu.VMEM((bpw,), jnp.int32),
            pltpu.VMEM((bpw,), jnp.int32),
            pltpu.VMEM((bpw,), jnp.float32),
            pltpu.VMEM((bpw,), jnp.float32),
            pltpu.VMEM((bpw,), jnp.float32),
            pltpu.VMEM((bpw,), jnp.float32),
            pltpu.VMEM((bpw,), jnp.float32),
            pltpu.SemaphoreType.DMA,
            pltpu.SemaphoreType.DMA,
        ],
    )
    return k(user, item,
             jnp.reshape(theta_table, (-1,)),
             jnp.reshape(a_table, (-1,)),
             jnp.reshape(b_table, (-1,)),
             jnp.reshape(c_table, (-1,)))


# speedup vs baseline: 1.1759x; 1.0391x over previous
import functools

import jax
import jax.numpy as jnp
from jax import lax
from jax.experimental import pallas as pl
from jax.experimental.pallas import tpu as pltpu
from jax.experimental.pallas import tpu_sc as plsc

BATCH = 16384
L = 16
D = 1.702


def _sigmoid(x):
    return 1.0 / (1.0 + jnp.exp(-x))


def _abc_body(nc, bpw, item_hbm, a_hbm, b_hbm, c_hbm,
              p_hbm, q_hbm, ct_hbm,
              iidx_v, a_v, b_v, c_v, p_v, q_v, ct_v, isem, gsem):
    wid = lax.axis_index("s") * nc + lax.axis_index("c")
    base = wid * bpw
    pltpu.sync_copy(item_hbm.at[pl.ds(base, bpw)], iidx_v)
    ca = pltpu.async_copy(a_hbm.at[iidx_v], a_v, gsem)
    cb = pltpu.async_copy(b_hbm.at[iidx_v], b_v, gsem)
    cc = pltpu.async_copy(c_hbm.at[iidx_v], c_v, gsem)
    ca.wait()
    cb.wait()
    cc.wait()

    def step(i, carry):
        sl = pl.ds(i * L, L)
        p = D * _sigmoid(a_v[sl])
        b = _sigmoid(b_v[sl]) - 0.5
        p_v[sl] = p
        q_v[sl] = p * b
        ct_v[sl] = _sigmoid(c_v[sl])
        return carry

    lax.fori_loop(0, bpw // L, step, 0)
    co1 = pltpu.async_copy(p_v, p_hbm.at[pl.ds(base, bpw)], isem)
    co2 = pltpu.async_copy(q_v, q_hbm.at[pl.ds(base, bpw)], isem)
    co3 = pltpu.async_copy(ct_v, ct_hbm.at[pl.ds(base, bpw)], isem)
    co1.wait()
    co2.wait()
    co3.wait()


def _theta_body(nc, bpw, user_hbm, theta_hbm, p_hbm, q_hbm, ct_hbm,
                out_hbm, uidx_v, th_v, p_v, q_v, ct_v, out_v, isem, gsem):
    wid = lax.axis_index("s") * nc + lax.axis_index("c")
    base = wid * bpw
    pltpu.sync_copy(user_hbm.at[pl.ds(base, bpw)], uidx_v)
    cth = pltpu.async_copy(theta_hbm.at[uidx_v], th_v, gsem)
    cp = pltpu.async_copy(p_hbm.at[pl.ds(base, bpw)], p_v, isem)
    cq = pltpu.async_copy(q_hbm.at[pl.ds(base, bpw)], q_v, isem)
    cct = pltpu.async_copy(ct_hbm.at[pl.ds(base, bpw)], ct_v, isem)
    cth.wait()
    cp.wait()
    cq.wait()
    cct.wait()

    def step(i, carry):
        sl = pl.ds(i * L, L)
        th = _sigmoid(th_v[sl]) - 0.5
        ct = ct_v[sl]
        out_v[sl] = ct + (1.0 - ct) * _sigmoid(p_v[sl] * th - q_v[sl])
        return carry

    lax.fori_loop(0, bpw // L, step, 0)
    pltpu.sync_copy(out_v, out_hbm.at[pl.ds(base, bpw)])


def kernel(user, item, theta_table, a_table, b_table, c_table):
    info = plsc.get_sparse_core_info()
    nc, ns = info.num_cores, info.num_subcores
    bpw = BATCH // (nc * ns)
    mesh = plsc.VectorSubcoreMesh(core_axis_name="c", subcore_axis_name="s")
    fvec = jax.ShapeDtypeStruct((BATCH,), jnp.float32)
    k1 = pl.kernel(
        functools.partial(_abc_body, nc, bpw),
        mesh=mesh,
        out_type=(fvec, fvec, fvec),
        scratch_types=[
            pltpu.VMEM((bpw,), jnp.int32),
            pltpu.VMEM((bpw,), jnp.float32),
            pltpu.VMEM((bpw,), jnp.float32),
            pltpu.VMEM((bpw,), jnp.float32),
            pltpu.VMEM((bpw,), jnp.float32),
            pltpu.VMEM((bpw,), jnp.float32),
            pltpu.VMEM((bpw,), jnp.float32),
            pltpu.SemaphoreType.DMA,
            pltpu.SemaphoreType.DMA,
        ],
    )
    k2 = pl.kernel(
        functools.partial(_theta_body, nc, bpw),
        mesh=mesh,
        out_type=fvec,
        scratch_types=[
            pltpu.VMEM((bpw,), jnp.int32),
            pltpu.VMEM((bpw,), jnp.float32),
            pltpu.VMEM((bpw,), jnp.float32),
            pltpu.VMEM((bpw,), jnp.float32),
            pltpu.VMEM((bpw,), jnp.float32),
            pltpu.VMEM((bpw,), jnp.float32),
            pltpu.SemaphoreType.DMA,
            pltpu.SemaphoreType.DMA,
        ],
    )
    p, q, ct = k1(item,
                  jnp.reshape(a_table, (-1,)),
                  jnp.reshape(b_table, (-1,)),
                  jnp.reshape(c_table, (-1,)))
    return k2(user, jnp.reshape(theta_table, (-1,)), p, q, ct)
